# Initial kernel scaffold; baseline (speedup 1.0000x reference)
import jax
import jax.numpy as jnp
from jax.experimental import pallas as pl


def kernel(input_ids, unigram):
    raise NotImplementedError("write your pallas kernel here")


# speedup vs baseline: 1.0007x; 1.0007x over previous
import functools

import jax
import jax.numpy as jnp
from jax import lax
from jax.experimental import pallas as pl
from jax.experimental.pallas import tpu as pltpu
from jax.experimental.pallas import tpu_sc as plsc

_NC = 2
_NS = 16
_NW = _NC * _NS

_C = 40


def kernel(input_ids, unigram):
    batch, hist = input_ids.shape
    _, dim = unigram.shape
    n = batch * hist
    per_w = n // _NW
    n_chunks = per_w // _C
    assert per_w % _C == 0 and n % _NW == 0
    idx = input_ids.reshape(n).astype(jnp.int32)

    mesh = plsc.VectorSubcoreMesh(core_axis_name="c", subcore_axis_name="s")

    @functools.partial(
        pl.kernel,
        out_type=jax.ShapeDtypeStruct((n, dim), unigram.dtype),
        mesh=mesh,
        compiler_params=pltpu.CompilerParams(use_tc_tiling_on_sc=False),
        scratch_types=[
            pltpu.VMEM((per_w,), jnp.int32),
            pltpu.VMEM((_C, dim), jnp.float32),
            pltpu.SemaphoreType.DMA,
        ],
    )
    def gather_kernel(table_hbm, idx_hbm, out_hbm, idx_v, rows_v, sem):
        wid = lax.axis_index("s") * _NC + lax.axis_index("c")
        base = wid * per_w
        pltpu.sync_copy(idx_hbm.at[pl.ds(base, per_w)], idx_v)

        @pl.loop(0, n_chunks)
        def _(c):
            off = c * _C
            pltpu.async_copy(
                table_hbm.at[idx_v.at[pl.ds(off, _C)]], rows_v, sem
            ).wait()
            pltpu.sync_copy(rows_v, out_hbm.at[pl.ds(base + off, _C)])

    out = gather_kernel(unigram, idx)
    return out.reshape(batch, hist, dim)


# speedup vs baseline: 1.0346x; 1.0339x over previous
import functools

import jax
import jax.numpy as jnp
from jax import lax
from jax.experimental import pallas as pl
from jax.experimental.pallas import tpu as pltpu
from jax.experimental.pallas import tpu_sc as plsc

_NC = 2
_NS = 16
_NW = _NC * _NS

_C = 40


def kernel(input_ids, unigram):
    batch, hist = input_ids.shape
    _, dim = unigram.shape
    n = batch * hist
    per_w = n // _NW
    n_chunks = per_w // _C
    assert per_w % _C == 0 and n % _NW == 0
    idx = input_ids.reshape(n).astype(jnp.int32)

    mesh = plsc.VectorSubcoreMesh(core_axis_name="c", subcore_axis_name="s")

    @functools.partial(
        pl.kernel,
        out_type=jax.ShapeDtypeStruct((n, dim), unigram.dtype),
        mesh=mesh,
        compiler_params=pltpu.CompilerParams(use_tc_tiling_on_sc=False),
        scratch_types=[
            pltpu.VMEM((per_w,), jnp.int32),
            pltpu.VMEM((2, _C, dim), jnp.float32),
            pltpu.SemaphoreType.DMA,
            pltpu.SemaphoreType.DMA,
            pltpu.SemaphoreType.DMA,
        ],
    )
    def gather_kernel(table_hbm, idx_hbm, out_hbm, idx_v, rows_v, gsem, wa, wb):
        wid = lax.axis_index("s") * _NC + lax.axis_index("c")
        base = wid * per_w
        pltpu.sync_copy(idx_hbm.at[pl.ds(base, per_w)], idx_v)
        wsems = (wa, wb)

        def gather_chunk(c, buf):
            pltpu.async_copy(
                table_hbm.at[idx_v.at[pl.ds(c * _C, _C)]], rows_v.at[buf], gsem
            ).wait()

        def start_write(c, buf):
            pltpu.async_copy(
                rows_v.at[buf], out_hbm.at[pl.ds(base + c * _C, _C)], wsems[buf]
            )

        def wait_write(buf):
            pltpu.make_async_copy(
                rows_v.at[buf], out_hbm.at[pl.ds(base, _C)], wsems[buf]
            ).wait()

        gather_chunk(0, 0)
        start_write(0, 0)
        gather_chunk(1, 1)
        start_write(1, 1)

        @pl.loop(2, n_chunks, step=2)
        def _(c):
            wait_write(0)
            gather_chunk(c, 0)
            start_write(c, 0)
            wait_write(1)
            gather_chunk(c + 1, 1)
            start_write(c + 1, 1)

        wait_write(0)
        wait_write(1)

    out = gather_kernel(unigram, idx)
    return out.reshape(batch, hist, dim)
